# baseline (speedup 1.0000x reference)
import functools

import jax
import jax.numpy as jnp
from jax import lax
from jax.experimental import pallas as pl
from jax.experimental.pallas import tpu as pltpu
from jax.experimental.pallas import tpu_sc as plsc

V = 1000
D = 1000
B = 1024
T = 200
N = B * T
NC = 2
NS = 16
NW = NC * NS
PER_W = N // NW
CH = 40
NCH = PER_W // CH


def _sc_gather(table, idx):
    mesh = plsc.VectorSubcoreMesh(core_axis_name="c", subcore_axis_name="s")

    @functools.partial(
        pl.kernel,
        mesh=mesh,
        out_type=jax.ShapeDtypeStruct((N, D), jnp.float32),
        scratch_types=[
            pltpu.VMEM((PER_W,), jnp.int32),
            pltpu.VMEM((CH, D), jnp.float32),
            pltpu.SemaphoreType.DMA,
        ],
        compiler_params=pltpu.CompilerParams(use_tc_tiling_on_sc=False),
    )
    def k(table_hbm, idx_hbm, out_hbm, idx_v, rows_v, gsem):
        wid = lax.axis_index("s") * NC + lax.axis_index("c")
        base = wid * PER_W
        pltpu.sync_copy(idx_hbm.at[pl.ds(base, PER_W)], idx_v)

        def body(g, carry):
            off = pl.multiple_of(g * CH, 8)
            pltpu.async_copy(
                table_hbm.at[idx_v.at[pl.ds(off, CH)]], rows_v, gsem
            ).wait()
            pltpu.sync_copy(rows_v, out_hbm.at[pl.ds(base + off, CH)])
            return carry

        lax.fori_loop(0, NCH, body, 0)

    return k(table, idx)


def kernel(encoding, table):
    idx = encoding.reshape(-1).astype(jnp.int32)
    out = _sc_gather(table, idx)
    return out.reshape(B, T, D)


# speedup vs baseline: 1.0471x; 1.0471x over previous
import functools

import jax
import jax.numpy as jnp
from jax import lax
from jax.experimental import pallas as pl
from jax.experimental.pallas import tpu as pltpu
from jax.experimental.pallas import tpu_sc as plsc

V = 1000
D = 1000
B = 1024
T = 200
N = B * T
NC = 2
NS = 16
NW = NC * NS
PER_W = N // NW
CH = 40
NCH = PER_W // CH


def _sc_gather(table, idx):
    mesh = plsc.VectorSubcoreMesh(core_axis_name="c", subcore_axis_name="s")

    @functools.partial(
        pl.kernel,
        mesh=mesh,
        out_type=jax.ShapeDtypeStruct((N, D), jnp.float32),
        scratch_types=[
            pltpu.VMEM((PER_W,), jnp.int32),
            pltpu.VMEM((2, CH, D), jnp.float32),
            pltpu.SemaphoreType.DMA,
        ],
        compiler_params=pltpu.CompilerParams(use_tc_tiling_on_sc=False),
    )
    def k(table_hbm, idx_hbm, out_hbm, idx_v, rows_v, gsem):
        wid = lax.axis_index("s") * NC + lax.axis_index("c")
        base = wid * PER_W
        pltpu.sync_copy(idx_hbm.at[pl.ds(base, PER_W)], idx_v)

        def start_gather(g, b):
            off = pl.multiple_of(g * CH, 8)
            pltpu.async_copy(
                table_hbm.at[idx_v.at[pl.ds(off, CH)]], rows_v.at[b], gsem
            )

        def wait_gather(b):
            pltpu.make_async_copy(
                table_hbm.at[pl.ds(0, CH)], rows_v.at[b], gsem
            ).wait()

        start_gather(0, 0)

        def body(i, carry):
            g0 = 2 * i
            start_gather(g0 + 1, 1)
            wait_gather(0)
            pltpu.sync_copy(rows_v.at[0], out_hbm.at[pl.ds(base + g0 * CH, CH)])

            @pl.when(g0 + 2 < NCH)
            def _():
                start_gather(g0 + 2, 0)

            wait_gather(1)
            pltpu.sync_copy(
                rows_v.at[1], out_hbm.at[pl.ds(base + (g0 + 1) * CH, CH)]
            )
            return carry

        lax.fori_loop(0, NCH // 2, body, 0)

    return k(table, idx)


def kernel(encoding, table):
    idx = encoding.reshape(-1).astype(jnp.int32)
    out = _sc_gather(table, idx)
    return out.reshape(B, T, D)
